# Initial kernel scaffold; baseline (speedup 1.0000x reference)
import jax
import jax.numpy as jnp
from jax.experimental import pallas as pl


def kernel(state, edge_index, action, Wg, bg, W1, b1, W2, b2, W3, b3):
    raise NotImplementedError("write your pallas kernel here")


# speedup vs baseline: 33.9788x; 33.9788x over previous
import functools

import jax
import jax.numpy as jnp
from jax import lax
from jax.experimental import pallas as pl
from jax.experimental.pallas import tpu as pltpu
from jax.experimental.pallas import tpu_sc as plsc

N = 10000
D = 128
E = 320000
NC, NS = 2, 16
NW = NC * NS
EPW = E // NW
CH = 80
NCH = EPW // CH
NPAD = 10240
ZROW = NPAD // NS
RPT = 624
ZB = 104
HID = 256
G = N // 8


def _mesh():
    return plsc.VectorSubcoreMesh(
        core_axis_name="c", subcore_axis_name="s", num_cores=NC, num_subcores=NS
    )


def _sc_hist_body(dst_hbm, degp_hbm, idx_v, ones_v, zrow_v, deg_sh):
    c = lax.axis_index("c")
    s = lax.axis_index("s")
    wid = s * NC + c

    for i in range(CH // 16):
        ones_v[pl.ds(i * 16, 16)] = jnp.full((16,), 1.0, jnp.float32)
    for i in range(ZROW // 16):
        zrow_v[pl.ds(i * 16, 16)] = jnp.zeros((16,), jnp.float32)

    pltpu.sync_copy(zrow_v, deg_sh.at[pl.ds(s * ZROW, ZROW)])
    plsc.subcore_barrier()

    pltpu.sync_copy(dst_hbm.at[wid], idx_v)

    def chunk(j, carry):
        pltpu.sync_copy(ones_v, deg_sh.at[idx_v.at[j]], add=True)
        return carry

    lax.fori_loop(0, NCH, chunk, 0)
    plsc.subcore_barrier()

    pltpu.sync_copy(deg_sh.at[pl.ds(s * ZROW, ZROW)],
                    degp_hbm.at[c, pl.ds(s * ZROW, ZROW)])


def _sc_hist(dst_r):
    fn = pl.kernel(
        _sc_hist_body,
        out_type=jax.ShapeDtypeStruct((NC, NPAD), jnp.float32),
        mesh=_mesh(),
        scratch_types=[
            pltpu.VMEM((NCH, CH), jnp.int32),
            pltpu.VMEM((CH,), jnp.float32),
            pltpu.VMEM((ZROW,), jnp.float32),
            pltpu.VMEM_SHARED((NPAD,), jnp.float32),
        ],
    )
    return fn(dst_r)


def _sc_scatter_body(edges_hbm, y_hbm, accp_hbm, dst_v, src0, src1,
                     rows0, rows1, acc_sh, semi0, semi1, sem0, sem1):
    c = lax.axis_index("c")
    s = lax.axis_index("s")
    wid = s * NC + c

    def zrow(j, carry):
        for k in range(D // 16):
            rows0[j, pl.ds(k * 16, 16)] = jnp.zeros((16,), jnp.float32)
        return carry

    lax.fori_loop(0, CH, zrow, 0)
    for i in range(7):
        pltpu.sync_copy(rows0, acc_sh.at[pl.ds(s * RPT + i * CH, CH)])
    pltpu.sync_copy(rows0.at[pl.ds(0, 64)],
                    acc_sh.at[pl.ds(s * RPT + 7 * CH, 64)])

    @pl.when(s == NS - 1)
    def _():
        pltpu.sync_copy(rows0.at[pl.ds(0, 16)], acc_sh.at[pl.ds(NS * RPT, 16)])

    plsc.subcore_barrier()

    pltpu.sync_copy(edges_hbm.at[1, wid], dst_v)

    bufs = ((src0, semi0, rows0, sem0), (src1, semi1, rows1, sem1))

    def _step(j, cur, nxt, has_next, has_next2):
        sb_c, sa_c, rb_c, sr_c = cur
        sb_n, sa_n, rb_n, sr_n = nxt
        if has_next:
            pltpu.make_async_copy(
                edges_hbm.at[0, wid, pl.ds(j + 1, 1)], sb_n, sa_n).wait()
            pltpu.async_copy(y_hbm.at[sb_n.at[0]], rb_n, sr_n)
        pltpu.make_async_copy(y_hbm.at[sb_c.at[0]], rb_c, sr_c).wait()
        if has_next2:
            pltpu.async_copy(
                edges_hbm.at[0, wid, pl.ds(jnp.minimum(j + 2, NCH - 1), 1)],
                sb_c, sa_c)
        pltpu.sync_copy(rb_c, acc_sh.at[dst_v.at[j]], add=True)

    pltpu.async_copy(edges_hbm.at[0, wid, pl.ds(0, 1)], src0, semi0).wait()
    pltpu.async_copy(y_hbm.at[src0.at[0]], rows0, sem0)
    pltpu.async_copy(edges_hbm.at[0, wid, pl.ds(1, 1)], src1, semi1)

    def run2(i, carry):
        j = 2 * i
        _step(j, bufs[0], bufs[1], True, True)
        _step(j + 1, bufs[1], bufs[0], True, True)
        return carry

    lax.fori_loop(0, (NCH - 1) // 2, run2, 0)
    pltpu.make_async_copy(
        edges_hbm.at[0, wid, pl.ds(NCH - 1, 1)], src1, semi1).wait()
    _step(NCH - 1, bufs[0], bufs[1], False, False)

    plsc.subcore_barrier()
    pltpu.sync_copy(acc_sh.at[pl.ds(s * RPT, RPT)],
                    accp_hbm.at[c, pl.ds(s * RPT, RPT)])

    @pl.when(s == NS - 1)
    def _():
        pltpu.sync_copy(acc_sh.at[pl.ds(NS * RPT, 16)],
                        accp_hbm.at[c, pl.ds(NS * RPT, 16)])


def _sc_scatter(edges_r, y):
    fn = pl.kernel(
        _sc_scatter_body,
        out_type=jax.ShapeDtypeStruct((NC, N, D), jnp.float32),
        mesh=_mesh(),
        scratch_types=[
            pltpu.VMEM((NCH, CH), jnp.int32),
            pltpu.VMEM((1, CH), jnp.int32),
            pltpu.VMEM((1, CH), jnp.int32),
            pltpu.VMEM((CH, D), jnp.float32),
            pltpu.VMEM((CH, D), jnp.float32),
            pltpu.VMEM_SHARED((N, D), jnp.float32),
            pltpu.SemaphoreType.DMA,
            pltpu.SemaphoreType.DMA,
            pltpu.SemaphoreType.DMA,
            pltpu.SemaphoreType.DMA,
        ],
    )
    return fn(edges_r, y)


def _tc_pre_body(state_ref, wg_ref, degp_ref, y_ref):
    deg = degp_ref[0] + degp_ref[1] + 1.0
    dis = lax.rsqrt(deg)
    xw = jnp.dot(state_ref[...], wg_ref[...], preferred_element_type=jnp.float32)
    y_ref[...] = xw * dis


def _tc_pre(state, Wg, degp3):
    blk = 1000
    grid = N // blk
    return pl.pallas_call(
        _tc_pre_body,
        grid=(grid,),
        in_specs=[
            pl.BlockSpec((blk, D), lambda i: (i, 0)),
            pl.BlockSpec((D, D), lambda i: (0, 0)),
            pl.BlockSpec((NC, blk, 1), lambda i: (0, i, 0)),
        ],
        out_specs=pl.BlockSpec((blk, D), lambda i: (i, 0)),
        out_shape=jax.ShapeDtypeStruct((N, D), jnp.float32),
        compiler_params=pltpu.CompilerParams(
            dimension_semantics=("parallel",)),
    )(state, Wg, degp3)


def _tc_post_body(accp_ref, y_ref, st_ref, degp_ref, act_ref, bg_ref,
                  w1_ref, b1_ref, w2_ref, b2_ref, w3_ref, b3_ref, q_ref):
    xr = jnp.zeros((G, D), jnp.float32)
    for k in range(8):
        sl = pl.ds(k * D, D)
        deg = degp_ref[0, :, k:k + 1] + degp_ref[1, :, k:k + 1] + 1.0
        dis = lax.rsqrt(deg)
        acc = accp_ref[0, :, sl] + accp_ref[1, :, sl] + y_ref[:, sl]
        z = jnp.maximum(acc * dis + bg_ref[...], 0.0) + st_ref[:, sl]
        xr = xr + z * (act_ref[:, k:k + 1] * 10.0)
    h = jnp.dot(xr, w1_ref[...], preferred_element_type=jnp.float32) + b1_ref[...]
    h = jnp.maximum(h, 0.0)
    h = jnp.dot(h, w2_ref[...], preferred_element_type=jnp.float32) + b2_ref[...]
    h = jnp.maximum(h, 0.0)
    q_ref[...] = jnp.dot(h, w3_ref[...], preferred_element_type=jnp.float32) + b3_ref[...]


def _tc_post(accp2, y2, st2, degp2, action, bg, W1, b1, W2, b2, W3, b3):
    return pl.pallas_call(
        _tc_post_body,
        out_shape=jax.ShapeDtypeStruct((G, 1), jnp.float32),
    )(accp2, y2, st2, degp2, action, bg.reshape(1, D),
      W1, b1.reshape(1, HID), W2, b2.reshape(1, HID), W3, b3.reshape(1, 1))


def kernel(state, edge_index, action, Wg, bg, W1, b1, W2, b2, W3, b3):
    edges_r = edge_index.reshape(2, NW, NCH, CH)

    degp = _sc_hist(edges_r[1])
    degp = degp[:, :N]

    y = _tc_pre(state, Wg, degp.reshape(NC, N, 1))

    accp = _sc_scatter(edges_r, y)

    q = _tc_post(
        accp.reshape(NC, G, 8 * D),
        y.reshape(G, 8 * D),
        state.reshape(G, 8 * D),
        degp.reshape(NC, G, 8),
        action, bg, W1, b1, W2, b2, W3, b3,
    )
    return q.reshape(G)


# speedup vs baseline: 35.8758x; 1.0558x over previous
import functools

import jax
import jax.numpy as jnp
from jax import lax
from jax.experimental import pallas as pl
from jax.experimental.pallas import tpu as pltpu
from jax.experimental.pallas import tpu_sc as plsc

N = 10000
D = 128
E = 320000
NC, NS = 2, 16
NW = NC * NS
EPW = E // NW
CH = 80
NCH = EPW // CH
NPAD = 10240
ZROW = NPAD // NS
RPT = 624
ZB = 104
HID = 256
G = N // 8


def _mesh():
    return plsc.VectorSubcoreMesh(
        core_axis_name="c", subcore_axis_name="s", num_cores=NC, num_subcores=NS
    )


def _sc_hist_body(dst_hbm, degp_hbm, idx_v, ones_v, zrow_v, deg_sh):
    c = lax.axis_index("c")
    s = lax.axis_index("s")
    wid = s * NC + c

    for i in range(CH // 16):
        ones_v[pl.ds(i * 16, 16)] = jnp.full((16,), 1.0, jnp.float32)
    for i in range(ZROW // 16):
        zrow_v[pl.ds(i * 16, 16)] = jnp.zeros((16,), jnp.float32)

    pltpu.sync_copy(zrow_v, deg_sh.at[pl.ds(s * ZROW, ZROW)])
    plsc.subcore_barrier()

    pltpu.sync_copy(dst_hbm.at[wid], idx_v)

    def chunk(j, carry):
        pltpu.sync_copy(ones_v, deg_sh.at[idx_v.at[j]], add=True)
        return carry

    lax.fori_loop(0, NCH, chunk, 0)
    plsc.subcore_barrier()

    pltpu.sync_copy(deg_sh.at[pl.ds(s * ZROW, ZROW)],
                    degp_hbm.at[c, pl.ds(s * ZROW, ZROW)])


def _sc_hist(dst_r):
    fn = pl.kernel(
        _sc_hist_body,
        out_type=jax.ShapeDtypeStruct((NC, NPAD), jnp.float32),
        mesh=_mesh(),
        scratch_types=[
            pltpu.VMEM((NCH, CH), jnp.int32),
            pltpu.VMEM((CH,), jnp.float32),
            pltpu.VMEM((ZROW,), jnp.float32),
            pltpu.VMEM_SHARED((NPAD,), jnp.float32),
        ],
    )
    return fn(dst_r)


def _sc_scatter_body(edges_hbm, y_hbm, accp_hbm, dst_v, src0, src1,
                     rows0, rows1, acc_sh, semi0, semi1, sem0, sem1):
    c = lax.axis_index("c")
    s = lax.axis_index("s")
    wid = s * NC + c

    def zrow(j, carry):
        for k in range(D // 16):
            rows0[j, pl.ds(k * 16, 16)] = jnp.zeros((16,), jnp.float32)
        return carry

    lax.fori_loop(0, CH, zrow, 0)
    for i in range(7):
        pltpu.sync_copy(rows0, acc_sh.at[pl.ds(s * RPT + i * CH, CH)])
    pltpu.sync_copy(rows0.at[pl.ds(0, 64)],
                    acc_sh.at[pl.ds(s * RPT + 7 * CH, 64)])

    @pl.when(s == NS - 1)
    def _():
        pltpu.sync_copy(rows0.at[pl.ds(0, 16)], acc_sh.at[pl.ds(NS * RPT, 16)])

    plsc.subcore_barrier()

    pltpu.sync_copy(edges_hbm.at[1, wid], dst_v)

    bufs = ((src0, semi0, rows0, sem0), (src1, semi1, rows1, sem1))

    def _step(j, cur, nxt, has_next, has_next2):
        sb_c, sa_c, rb_c, sr_c = cur
        sb_n, sa_n, rb_n, sr_n = nxt
        if has_next:
            pltpu.make_async_copy(
                edges_hbm.at[0, wid, pl.ds(j + 1, 1)], sb_n, sa_n).wait()
            pltpu.async_copy(y_hbm.at[sb_n.at[0]], rb_n, sr_n)
        pltpu.make_async_copy(y_hbm.at[sb_c.at[0]], rb_c, sr_c).wait()
        if has_next2:
            pltpu.async_copy(
                edges_hbm.at[0, wid, pl.ds(jnp.minimum(j + 2, NCH - 1), 1)],
                sb_c, sa_c)
        pltpu.sync_copy(rb_c, acc_sh.at[dst_v.at[j]], add=True)

    pltpu.async_copy(edges_hbm.at[0, wid, pl.ds(0, 1)], src0, semi0).wait()
    pltpu.async_copy(y_hbm.at[src0.at[0]], rows0, sem0)
    pltpu.async_copy(edges_hbm.at[0, wid, pl.ds(1, 1)], src1, semi1)

    def run2(i, carry):
        j = 2 * i
        _step(j, bufs[0], bufs[1], True, True)
        _step(j + 1, bufs[1], bufs[0], True, True)
        return carry

    lax.fori_loop(0, (NCH - 1) // 2, run2, 0)
    pltpu.make_async_copy(
        edges_hbm.at[0, wid, pl.ds(NCH - 1, 1)], src1, semi1).wait()
    _step(NCH - 1, bufs[0], bufs[1], False, False)

    plsc.subcore_barrier()
    pltpu.sync_copy(acc_sh.at[pl.ds(s * RPT, RPT)],
                    accp_hbm.at[c, pl.ds(s * RPT, RPT)])

    @pl.when(s == NS - 1)
    def _():
        pltpu.sync_copy(acc_sh.at[pl.ds(NS * RPT, 16)],
                        accp_hbm.at[c, pl.ds(NS * RPT, 16)])


def _sc_scatter(edges_r, y):
    fn = pl.kernel(
        _sc_scatter_body,
        out_type=jax.ShapeDtypeStruct((NC, N, D), jnp.float32),
        mesh=_mesh(),
        scratch_types=[
            pltpu.VMEM((NCH, CH), jnp.int32),
            pltpu.VMEM((1, CH), jnp.int32),
            pltpu.VMEM((1, CH), jnp.int32),
            pltpu.VMEM((CH, D), jnp.float32),
            pltpu.VMEM((CH, D), jnp.float32),
            pltpu.VMEM_SHARED((N, D), jnp.float32),
            pltpu.SemaphoreType.DMA,
            pltpu.SemaphoreType.DMA,
            pltpu.SemaphoreType.DMA,
            pltpu.SemaphoreType.DMA,
        ],
    )
    return fn(edges_r, y)


def _tc_pre_body(state_ref, wg_ref, degp_ref, y_ref):
    deg = degp_ref[0] + degp_ref[1] + 1.0
    dis = lax.rsqrt(deg)
    xw = jnp.dot(state_ref[...], wg_ref[...], preferred_element_type=jnp.float32)
    y_ref[...] = xw * dis


def _tc_pre(state, Wg, degp3):
    blk = 1000
    grid = N // blk
    return pl.pallas_call(
        _tc_pre_body,
        grid=(grid,),
        in_specs=[
            pl.BlockSpec((blk, D), lambda i: (i, 0)),
            pl.BlockSpec((D, D), lambda i: (0, 0)),
            pl.BlockSpec((NC, blk, 1), lambda i: (0, i, 0)),
        ],
        out_specs=pl.BlockSpec((blk, D), lambda i: (i, 0)),
        out_shape=jax.ShapeDtypeStruct((N, D), jnp.float32),
        compiler_params=pltpu.CompilerParams(
            dimension_semantics=("parallel",)),
    )(state, Wg, degp3)


BLK = 1000
GB = BLK // 8


def _tc_post_body(accp_ref, y_ref, st_ref, degp_ref, act_ref, bg_ref,
                  w1_ref, b1_ref, w2_ref, b2_ref, w3_ref, b3_ref, q_ref):
    deg = degp_ref[0] + degp_ref[1] + 1.0
    dis = lax.rsqrt(deg)
    acc = accp_ref[0] + accp_ref[1] + y_ref[...]
    z = jnp.maximum(acc * dis + bg_ref[...], 0.0) + st_ref[...]
    rows = lax.broadcasted_iota(jnp.int32, (GB, BLK), 0)
    cols = lax.broadcasted_iota(jnp.int32, (GB, BLK), 1)
    sel = jnp.where(lax.shift_right_logical(cols, 3) == rows, 1.0, 0.0)
    S = sel * (act_ref[0] * 10.0)
    xr = jnp.dot(S, z, preferred_element_type=jnp.float32)
    h = jnp.dot(xr, w1_ref[...], preferred_element_type=jnp.float32) + b1_ref[...]
    h = jnp.maximum(h, 0.0)
    h = jnp.dot(h, w2_ref[...], preferred_element_type=jnp.float32) + b2_ref[...]
    h = jnp.maximum(h, 0.0)
    q_ref[0] = jnp.dot(h, w3_ref[...], preferred_element_type=jnp.float32) + b3_ref[...]


def _tc_post(accp, y, state, degp3, a3, bg, W1, b1, W2, b2, W3, b3):
    grid = N // BLK
    q = pl.pallas_call(
        _tc_post_body,
        grid=(grid,),
        in_specs=[
            pl.BlockSpec((NC, BLK, D), lambda i: (0, i, 0)),
            pl.BlockSpec((BLK, D), lambda i: (i, 0)),
            pl.BlockSpec((BLK, D), lambda i: (i, 0)),
            pl.BlockSpec((NC, BLK, 1), lambda i: (0, i, 0)),
            pl.BlockSpec((1, 1, BLK), lambda i: (i, 0, 0)),
            pl.BlockSpec((1, D), lambda i: (0, 0)),
            pl.BlockSpec((D, HID), lambda i: (0, 0)),
            pl.BlockSpec((1, HID), lambda i: (0, 0)),
            pl.BlockSpec((HID, HID), lambda i: (0, 0)),
            pl.BlockSpec((1, HID), lambda i: (0, 0)),
            pl.BlockSpec((HID, 1), lambda i: (0, 0)),
            pl.BlockSpec((1, 1), lambda i: (0, 0)),
        ],
        out_specs=pl.BlockSpec((1, GB, 1), lambda i: (i, 0, 0)),
        out_shape=jax.ShapeDtypeStruct((grid, GB, 1), jnp.float32),
        compiler_params=pltpu.CompilerParams(
            dimension_semantics=("parallel",)),
    )(accp, y, state, degp3, a3, bg.reshape(1, D),
      W1, b1.reshape(1, HID), W2, b2.reshape(1, HID), W3, b3.reshape(1, 1))
    return q.reshape(G)


def kernel(state, edge_index, action, Wg, bg, W1, b1, W2, b2, W3, b3):
    edges_r = edge_index.reshape(2, NW, NCH, CH)

    degp = _sc_hist(edges_r[1])
    degp = degp[:, :N]

    y = _tc_pre(state, Wg, degp.reshape(NC, N, 1))

    accp = _sc_scatter(edges_r, y)

    return _tc_post(
        accp, y, state,
        degp.reshape(NC, N, 1),
        action.reshape(N // BLK, 1, BLK),
        bg, W1, b1, W2, b2, W3, b3,
    )
